# baseline (speedup 1.0000x reference)
import functools

import jax
import jax.numpy as jnp
from jax import lax
from jax.experimental import pallas as pl
from jax.experimental.pallas import tpu as pltpu
from jax.experimental.pallas import tpu_sc as plsc

VOCAB = 100000
EMB = 128
NCLS = 6
B = 4096
L = 200

PW = 16
NC, NS = 2, 16
NW = NC * NS
BPW = B // NW
CH = 16
NCHUNK = BPW // CH
IDX_PER_CH = CH * L
DMA_ROWS = 128
NDMA = IDX_PER_CH // DMA_ROWS
TC_BLK = 4000

_UNROLL = 8


PACK = 128 // PW
NPACK = VOCAB // PACK
TC_ROWS = 512


def _tc_project_body(embr_ref, w1_ref, b1_ref, w2p_ref, b2p_ref, o_ref):
    hi = lax.Precision.HIGHEST
    wc = lax.dot_general(w2p_ref[...], w1_ref[...], (((1,), (0,)), ((), ())),
                         precision=hi, preferred_element_type=jnp.float32)
    c = lax.dot_general(b1_ref[...], w2p_ref[...], (((1,), (1,)), ((), ())),
                        precision=hi, preferred_element_type=jnp.float32)
    c = c + b2p_ref[...]
    wct = jnp.transpose(wc)
    row = jnp.concatenate([wct] * PACK, axis=1)
    big = jnp.concatenate([row] * PACK, axis=0)
    i0 = lax.broadcasted_iota(jnp.int32, (PACK * EMB, 128), 0) // EMB
    i1 = lax.broadcasted_iota(jnp.int32, (PACK * EMB, 128), 1) // PW
    wbig = jnp.where(i0 == i1, big, 0.0)
    p = lax.dot_general(embr_ref[...], wbig, (((1,), (0,)), ((), ())),
                        precision=hi, preferred_element_type=jnp.float32)
    ctile = jnp.concatenate([c] * PACK, axis=1)
    o_ref[...] = (p + ctile) * (1.0 / L)


def _project_table(emb_r, w1, b1, w2p, b2p):
    grid = (NPACK + TC_ROWS - 1) // TC_ROWS
    return pl.pallas_call(
        _tc_project_body,
        grid=(grid,),
        in_specs=[
            pl.BlockSpec((TC_ROWS, PACK * EMB), lambda i: (i, 0)),
            pl.BlockSpec((50, EMB), lambda i: (0, 0)),
            pl.BlockSpec((1, 50), lambda i: (0, 0)),
            pl.BlockSpec((PW, 50), lambda i: (0, 0)),
            pl.BlockSpec((1, PW), lambda i: (0, 0)),
        ],
        out_specs=pl.BlockSpec((TC_ROWS, 128), lambda i: (i, 0)),
        out_shape=jax.ShapeDtypeStruct((NPACK, 128), jnp.float32),
    )(emb_r, w1, b1, w2p, b2p)


def _sc_body(text_hbm, p_hbm, out_hbm, idx0, idx1, g0, g1, obuf, sem0, sem1):
    wid = lax.axis_index("s") * NC + lax.axis_index("c")
    bufs = ((idx0, g0, sem0), (idx1, g1, sem1))

    def load_chunk(k, ib, gb, sem):
        pltpu.sync_copy(text_hbm.at[wid, k], ib)
        return [
            pltpu.async_copy(p_hbm.at[ib.at[j]],
                             gb.at[pl.ds(j * DMA_ROWS, DMA_ROWS)], sem)
            for j in range(NDMA)
        ]

    def accum_chunk(gb, k):
        def body_b(b, carry):
            base = b * L

            def body_t(tt, accs):
                i = base + tt * _UNROLL
                a0, a1, a2, a3 = accs
                a0 = a0 + gb[i]
                a1 = a1 + gb[i + 1]
                a2 = a2 + gb[i + 2]
                a3 = a3 + gb[i + 3]
                a0 = a0 + gb[i + 4]
                a1 = a1 + gb[i + 5]
                a2 = a2 + gb[i + 6]
                a3 = a3 + gb[i + 7]
                return (a0, a1, a2, a3)

            z = jnp.zeros((PW,), jnp.float32)
            a0, a1, a2, a3 = lax.fori_loop(0, L // _UNROLL, body_t,
                                           (z, z, z, z))
            obuf[k * CH + b] = (a0 + a1) + (a2 + a3)
            return carry

        lax.fori_loop(0, CH, body_b, 0)

    descs = [None, None]
    descs[0] = load_chunk(0, *bufs[0])
    for k in range(NCHUNK):
        p = k % 2
        if k + 1 < NCHUNK:
            descs[1 - p] = load_chunk(k + 1, *bufs[1 - p])
        for d in descs[p]:
            d.wait()
        accum_chunk(bufs[p][1], k)

    pltpu.sync_copy(obuf, out_hbm.at[pl.ds(wid * BPW, BPW)])


_sc_gather_sum = functools.partial(
    pl.kernel,
    out_type=jax.ShapeDtypeStruct((B, PW), jnp.float32),
    mesh=plsc.VectorSubcoreMesh(core_axis_name="c", subcore_axis_name="s"),
    compiler_params=pltpu.CompilerParams(use_tc_tiling_on_sc=False),
    scratch_types=[
        pltpu.VMEM((NDMA, DMA_ROWS), jnp.int32),
        pltpu.VMEM((NDMA, DMA_ROWS), jnp.int32),
        pltpu.VMEM((IDX_PER_CH, PW), jnp.float32),
        pltpu.VMEM((IDX_PER_CH, PW), jnp.float32),
        pltpu.VMEM((BPW, PW), jnp.float32),
        pltpu.SemaphoreType.DMA,
        pltpu.SemaphoreType.DMA,
    ],
)(_sc_body)


def kernel(text, emb_table, W1, b1, W2, b2):
    w2p = jnp.zeros((PW, 50), jnp.float32).at[:NCLS].set(W2)
    b2p = jnp.zeros((1, PW), jnp.float32).at[0, :NCLS].set(b2)
    emb_r = emb_table.reshape(NPACK, PACK * EMB)
    p_packed = _project_table(emb_r, W1, b1.reshape(1, 50), w2p, b2p)
    p_table = p_packed.reshape(VOCAB, PW)
    text_lin = lax.optimization_barrier(text.astype(jnp.int32).reshape(-1))
    text_r = text_lin.reshape(NW, NCHUNK, NDMA, DMA_ROWS)
    out = _sc_gather_sum(text_r, p_table)
    return out[:, :NCLS]


# speedup vs baseline: 1.1296x; 1.1296x over previous
import functools

import jax
import jax.numpy as jnp
from jax import lax
from jax.experimental import pallas as pl
from jax.experimental.pallas import tpu as pltpu
from jax.experimental.pallas import tpu_sc as plsc

VOCAB = 100000
EMB = 128
NCLS = 6
B = 4096
L = 200

PW = 16
NC, NS = 2, 16
NW = NC * NS
BPW = B // NW
CH = 16
NCHUNK = BPW // CH
IDX_PER_CH = CH * L
DMA_ROWS = 128
NDMA = IDX_PER_CH // DMA_ROWS
TC_BLK = 4000

_UNROLL = 8


TC_BLK2 = 4096
VOCAB_PAD = 102400


def _tc_project_body(emb_ref, w1_ref, b1_ref, w2p_ref, b2p_ref, pt_ref):
    hi = lax.Precision.HIGHEST
    wc = lax.dot_general(w2p_ref[...], w1_ref[...], (((1,), (0,)), ((), ())),
                         precision=hi, preferred_element_type=jnp.float32)
    ct = lax.dot_general(w2p_ref[...], b1_ref[...], (((1,), (0,)), ((), ())),
                         precision=hi, preferred_element_type=jnp.float32)
    ct = ct + b2p_ref[...]
    pt = lax.dot_general(wc, emb_ref[...], (((1,), (1,)), ((), ())),
                         preferred_element_type=jnp.float32)
    pt_ref[...] = (pt + ct) * (1.0 / L)


def _project_table(emb_table, w1, b1, w2p, b2p):
    grid = VOCAB_PAD // TC_BLK2
    return pl.pallas_call(
        _tc_project_body,
        grid=(grid,),
        in_specs=[
            pl.BlockSpec((TC_BLK2, EMB), lambda i: (i, 0)),
            pl.BlockSpec((50, EMB), lambda i: (0, 0)),
            pl.BlockSpec((50, 1), lambda i: (0, 0)),
            pl.BlockSpec((PW, 50), lambda i: (0, 0)),
            pl.BlockSpec((PW, 1), lambda i: (0, 0)),
        ],
        out_specs=pl.BlockSpec((PW, TC_BLK2), lambda i: (0, i)),
        out_shape=jax.ShapeDtypeStruct((PW, VOCAB_PAD), jnp.float32),
    )(emb_table, w1, b1, w2p, b2p)


def _sc_body(text_hbm, p_hbm, out_hbm, idx0, idx1, g0, g1, obuf, sem0, sem1):
    wid = lax.axis_index("s") * NC + lax.axis_index("c")
    bufs = ((idx0, g0, sem0), (idx1, g1, sem1))

    def load_chunk(k, ib, gb, sem):
        pltpu.sync_copy(text_hbm.at[wid, k], ib)
        return [
            pltpu.async_copy(p_hbm.at[ib.at[j]],
                             gb.at[pl.ds(j * DMA_ROWS, DMA_ROWS)], sem)
            for j in range(NDMA)
        ]

    def accum_chunk(gb, k):
        def body_b(b, carry):
            base = b * L

            def body_t(tt, accs):
                i = base + tt * _UNROLL
                a0, a1, a2, a3 = accs
                a0 = a0 + gb[i]
                a1 = a1 + gb[i + 1]
                a2 = a2 + gb[i + 2]
                a3 = a3 + gb[i + 3]
                a0 = a0 + gb[i + 4]
                a1 = a1 + gb[i + 5]
                a2 = a2 + gb[i + 6]
                a3 = a3 + gb[i + 7]
                return (a0, a1, a2, a3)

            z = jnp.zeros((PW,), jnp.float32)
            a0, a1, a2, a3 = lax.fori_loop(0, L // _UNROLL, body_t,
                                           (z, z, z, z))
            obuf[k * CH + b] = (a0 + a1) + (a2 + a3)
            return carry

        lax.fori_loop(0, CH, body_b, 0)

    descs = [None, None]
    descs[0] = load_chunk(0, *bufs[0])
    for k in range(NCHUNK):
        p = k % 2
        if k + 1 < NCHUNK:
            descs[1 - p] = load_chunk(k + 1, *bufs[1 - p])
        for d in descs[p]:
            d.wait()
        accum_chunk(bufs[p][1], k)

    pltpu.sync_copy(obuf, out_hbm.at[pl.ds(wid * BPW, BPW)])


_sc_gather_sum = functools.partial(
    pl.kernel,
    out_type=jax.ShapeDtypeStruct((B, PW), jnp.float32),
    mesh=plsc.VectorSubcoreMesh(core_axis_name="c", subcore_axis_name="s"),
    compiler_params=pltpu.CompilerParams(use_tc_tiling_on_sc=False),
    scratch_types=[
        pltpu.VMEM((NDMA, DMA_ROWS), jnp.int32),
        pltpu.VMEM((NDMA, DMA_ROWS), jnp.int32),
        pltpu.VMEM((IDX_PER_CH, PW), jnp.float32),
        pltpu.VMEM((IDX_PER_CH, PW), jnp.float32),
        pltpu.VMEM((BPW, PW), jnp.float32),
        pltpu.SemaphoreType.DMA,
        pltpu.SemaphoreType.DMA,
    ],
)(_sc_body)


def kernel(text, emb_table, W1, b1, W2, b2):
    w2p = jnp.zeros((PW, 50), jnp.float32).at[:NCLS].set(W2)
    b2p = jnp.zeros((PW, 1), jnp.float32).at[:NCLS, 0].set(b2)
    p_t = _project_table(emb_table, W1, b1.reshape(50, 1), w2p, b2p)
    p_table = p_t.T
    text_lin = lax.optimization_barrier(text.astype(jnp.int32).reshape(-1))
    text_r = text_lin.reshape(NW, NCHUNK, NDMA, DMA_ROWS)
    out = _sc_gather_sum(text_r, p_table)
    return out[:, :NCLS]


# speedup vs baseline: 1.5530x; 1.3748x over previous
import functools

import jax
import jax.numpy as jnp
from jax import lax
from jax.experimental import pallas as pl
from jax.experimental.pallas import tpu as pltpu
from jax.experimental.pallas import tpu_sc as plsc

VOCAB = 100000
EMB = 128
NCLS = 6
B = 4096
L = 200

PW = 16
NC, NS = 2, 16
NW = NC * NS
BPW = B // NW
CH = 16
NCHUNK = BPW // CH
IDX_PER_CH = CH * L
DMA_ROWS = 128
NDMA = IDX_PER_CH // DMA_ROWS
TC_BLK = 4000

_UNROLL = 8


TC_BLK2 = 4096
VOCAB_PAD = 102400


def _tc_project_body(emb_ref, w1_ref, b1_ref, w2p_ref, b2p_ref, p_ref):
    hi = lax.Precision.HIGHEST
    wc = lax.dot_general(w2p_ref[...], w1_ref[...], (((1,), (0,)), ((), ())),
                         precision=hi, preferred_element_type=jnp.float32)
    c = lax.dot_general(b1_ref[...], w2p_ref[...], (((1,), (1,)), ((), ())),
                        precision=hi, preferred_element_type=jnp.float32)
    c = c + b2p_ref[...]
    wcp = jnp.concatenate(
        [jnp.transpose(wc), jnp.zeros((EMB, 128 - PW), jnp.float32)], axis=1)
    cp = jnp.concatenate([c, jnp.zeros((1, 128 - PW), jnp.float32)], axis=1)
    p = lax.dot_general(emb_ref[...], wcp, (((1,), (0,)), ((), ())),
                        preferred_element_type=jnp.float32)
    p_ref[...] = (p + cp) * (1.0 / L)


def _project_table(emb_table, w1, b1, w2p, b2p):
    grid = VOCAB_PAD // TC_BLK2
    return pl.pallas_call(
        _tc_project_body,
        grid=(grid,),
        in_specs=[
            pl.BlockSpec((TC_BLK2, EMB), lambda i: (i, 0)),
            pl.BlockSpec((50, EMB), lambda i: (0, 0)),
            pl.BlockSpec((1, 50), lambda i: (0, 0)),
            pl.BlockSpec((PW, 50), lambda i: (0, 0)),
            pl.BlockSpec((1, PW), lambda i: (0, 0)),
        ],
        out_specs=pl.BlockSpec((TC_BLK2, 128), lambda i: (i, 0)),
        out_shape=jax.ShapeDtypeStruct((VOCAB_PAD, 128), jnp.float32),
    )(emb_table, w1, b1, w2p, b2p)


def _sc_body(text_hbm, p_hbm, out_hbm, idx0, idx1, g0, g1, obuf, sem0, sem1):
    wid = lax.axis_index("s") * NC + lax.axis_index("c")
    bufs = ((idx0, g0, sem0), (idx1, g1, sem1))

    def load_chunk(k, ib, gb, sem):
        pltpu.sync_copy(text_hbm.at[wid, k], ib)
        return [
            pltpu.async_copy(p_hbm.at[ib.at[j]],
                             gb.at[pl.ds(j * DMA_ROWS, DMA_ROWS)], sem)
            for j in range(NDMA)
        ]

    def accum_chunk(gb, k):
        def body_b(b, carry):
            base = b * L

            def body_t(tt, accs):
                i = base + tt * _UNROLL
                a0, a1, a2, a3 = accs
                a0 = a0 + gb[i]
                a1 = a1 + gb[i + 1]
                a2 = a2 + gb[i + 2]
                a3 = a3 + gb[i + 3]
                a0 = a0 + gb[i + 4]
                a1 = a1 + gb[i + 5]
                a2 = a2 + gb[i + 6]
                a3 = a3 + gb[i + 7]
                return (a0, a1, a2, a3)

            z = jnp.zeros((PW,), jnp.float32)
            a0, a1, a2, a3 = lax.fori_loop(0, L // _UNROLL, body_t,
                                           (z, z, z, z))
            obuf[k * CH + b] = (a0 + a1) + (a2 + a3)
            return carry

        lax.fori_loop(0, CH, body_b, 0)

    descs = [None, None]
    descs[0] = load_chunk(0, *bufs[0])
    for k in range(NCHUNK):
        p = k % 2
        if k + 1 < NCHUNK:
            descs[1 - p] = load_chunk(k + 1, *bufs[1 - p])
        for d in descs[p]:
            d.wait()
        accum_chunk(bufs[p][1], k)

    pltpu.sync_copy(obuf, out_hbm.at[pl.ds(wid * BPW, BPW)])


_sc_gather_sum = functools.partial(
    pl.kernel,
    out_type=jax.ShapeDtypeStruct((B, PW), jnp.float32),
    mesh=plsc.VectorSubcoreMesh(core_axis_name="c", subcore_axis_name="s"),
    compiler_params=pltpu.CompilerParams(use_tc_tiling_on_sc=False),
    scratch_types=[
        pltpu.VMEM((NDMA, DMA_ROWS), jnp.int32),
        pltpu.VMEM((NDMA, DMA_ROWS), jnp.int32),
        pltpu.VMEM((IDX_PER_CH, PW), jnp.float32),
        pltpu.VMEM((IDX_PER_CH, PW), jnp.float32),
        pltpu.VMEM((BPW, PW), jnp.float32),
        pltpu.SemaphoreType.DMA,
        pltpu.SemaphoreType.DMA,
    ],
)(_sc_body)


def kernel(text, emb_table, W1, b1, W2, b2):
    w2p = jnp.zeros((PW, 50), jnp.float32).at[:NCLS].set(W2)
    b2p = jnp.zeros((1, PW), jnp.float32).at[0, :NCLS].set(b2)
    p_wide = _project_table(emb_table, W1, b1.reshape(1, 50), w2p, b2p)
    p_table = p_wide.reshape(8 * VOCAB_PAD, PW)
    text_lin = lax.optimization_barrier(
        text.astype(jnp.int32).reshape(-1) * 8)
    text_r = text_lin.reshape(NW, NCHUNK, NDMA, DMA_ROWS)
    out = _sc_gather_sum(text_r, p_table)
    return out[:, :NCLS]
